# baseline (speedup 1.0000x reference)
import functools

import jax
import jax.numpy as jnp
from jax import lax
from jax.experimental import pallas as pl
from jax.experimental.pallas import tpu as pltpu
from jax.experimental.pallas import tpu_sc as plsc

D = 64
B = 16384
NC = 2
NS = 16
L = 16
NW = NC * NS
BPW = B // NW
CHUNK = 128
NCHUNK = BPW // CHUNK
N_ENT = 1_000_000
N_REL = 1000
GRP = 16000
NGRP = -(-N_ENT // GRP)
P_ROWS = ((NGRP + 1) // 2) * GRP


def _eye(dtype):
    return (lax.broadcasted_iota(jnp.int32, (D, D), 0)
            == lax.broadcasted_iota(jnp.int32, (D, D), 1)).astype(dtype)


def _transpose_pack_entities(ent_t):
    def body(x0_ref, x1_ref, o_ref):
        eye = _eye(jnp.bfloat16)
        y0 = lax.dot_general(x0_ref[...].astype(jnp.bfloat16), eye,
                             (((0,), (0,)), ((), ())),
                             preferred_element_type=jnp.float32)
        y1 = lax.dot_general(x1_ref[...].astype(jnp.bfloat16), eye,
                             (((0,), (0,)), ((), ())),
                             preferred_element_type=jnp.float32)
        o_ref[:, 0:D] = y0
        o_ref[:, D:2 * D] = y1

    sub = GRP // 5
    nin = -(-N_ENT // sub)
    return pl.pallas_call(
        body,
        grid=((NGRP + 1) // 2, 5),
        in_specs=[
            pl.BlockSpec((D, sub),
                         lambda c, h: (0, jnp.minimum(10 * c + h, nin - 1))),
            pl.BlockSpec((D, sub),
                         lambda c, h: (0, jnp.minimum(10 * c + 5 + h, nin - 1))),
        ],
        out_specs=pl.BlockSpec((sub, 2 * D), lambda c, h: (5 * c + h, 0)),
        out_shape=jax.ShapeDtypeStruct((P_ROWS, 2 * D), jnp.float32),
    )(ent_t, ent_t)


def _transpose_pad_relations(rel_t):
    def body(x_ref, o_ref):
        y = lax.dot_general(x_ref[...], _eye(jnp.float32),
                            (((0,), (0,)), ((), ())),
                            preferred_element_type=jnp.float32)
        o_ref[:, 0:D] = y
        o_ref[:, D:2 * D] = jnp.zeros_like(y)

    return pl.pallas_call(
        body,
        out_shape=jax.ShapeDtypeStruct((N_REL, 2 * D), jnp.float32),
    )(rel_t)


def _sc_gather_score(subj, rel, obj, p_ent, p_rel):
    mesh = plsc.VectorSubcoreMesh(core_axis_name="c", subcore_axis_name="s")

    @functools.partial(
        pl.kernel,
        mesh=mesh,
        out_type=jax.ShapeDtypeStruct((B,), jnp.float32),
        compiler_params=pltpu.CompilerParams(needs_layout_passes=False),
        scratch_types=[
            pltpu.VMEM((BPW,), jnp.int32),
            pltpu.VMEM((BPW,), jnp.int32),
            pltpu.VMEM((BPW,), jnp.int32),
            pltpu.VMEM((BPW,), jnp.int32),
            pltpu.VMEM((BPW,), jnp.int32),
            pltpu.VMEM((CHUNK, 2 * D), jnp.float32),
            pltpu.VMEM((CHUNK, 2 * D), jnp.float32),
            pltpu.VMEM((CHUNK, 2 * D), jnp.float32),
            pltpu.VMEM((BPW,), jnp.float32),
            pltpu.SemaphoreType.DMA,
        ],
    )
    def k(subj_h, rel_h, obj_h, ent_h, relf_h, out_h,
          sidx, ridx, oidx, soff, ooff, srows, rrows, orows, outv, sem):
        wid = lax.axis_index("s") * NC + lax.axis_index("c")
        base = wid * BPW
        pltpu.sync_copy(subj_h.at[pl.ds(base, BPW)], sidx)
        pltpu.sync_copy(rel_h.at[pl.ds(base, BPW)], ridx)
        pltpu.sync_copy(obj_h.at[pl.ds(base, BPW)], oidx)

        def xform(v, idx_ref, off_ref, i):
            g = v // GRP
            p = (g >> 1) * GRP + (v - g * GRP)
            idx_ref[pl.ds(i * L, L)] = p
            off_ref[pl.ds(i * L, L)] = (g & 1) * D

        def xf_body(i, carry):
            sl = pl.ds(i * L, L)
            xform(sidx[sl], sidx, soff, i)
            xform(oidx[sl], oidx, ooff, i)
            return carry

        lax.fori_loop(0, BPW // L, xf_body, 0)

        iota = lax.iota(jnp.int32, L)

        for c in range(NCHUNK):
            isl = pl.ds(c * CHUNK, CHUNK)
            cs = pltpu.async_copy(ent_h.at[sidx.at[isl]], srows, sem)
            cr = pltpu.async_copy(relf_h.at[ridx.at[isl]], rrows, sem)
            co = pltpu.async_copy(ent_h.at[oidx.at[isl]], orows, sem)
            cs.wait()
            cr.wait()
            co.wait()

            def g_body(g, carry):
                base_p = g * L
                sofv = soff[pl.ds(c * CHUNK + base_p, L)]
                oofv = ooff[pl.ds(c * CHUNK + base_p, L)]
                acc_vec = jnp.zeros((L,), jnp.float32)
                for j in range(L):
                    p = base_p + j
                    s_hi = sofv[j] != 0
                    o_hi = oofv[j] != 0
                    acc = None
                    for kk in range(D // L):
                        sv = jnp.where(s_hi,
                                       srows[p, pl.ds(D + kk * L, L)],
                                       srows[p, pl.ds(kk * L, L)])
                        rv = rrows[p, pl.ds(kk * L, L)]
                        ov = jnp.where(o_hi,
                                       orows[p, pl.ds(D + kk * L, L)],
                                       orows[p, pl.ds(kk * L, L)])
                        t = sv * rv * ov
                        acc = t if acc is None else acc + t
                    acc_vec = jnp.where(iota == j, jnp.sum(acc), acc_vec)
                outv[pl.ds(c * CHUNK + base_p, L)] = acc_vec
                return carry

            lax.fori_loop(0, CHUNK // L, g_body, 0)

        pltpu.sync_copy(outv, out_h.at[pl.ds(base, BPW)])

    return k(subj, rel, obj, p_ent, p_rel)


def kernel(subjects, relations, objects, entity_factors, relations_factors):
    p_ent = _transpose_pack_entities(entity_factors.T)
    p_rel = _transpose_pad_relations(relations_factors.T)
    return _sc_gather_score(subjects, relations, objects, p_ent, p_rel)


# speedup vs baseline: 1.3600x; 1.3600x over previous
import functools

import jax
import jax.numpy as jnp
from jax import lax
from jax.experimental import pallas as pl
from jax.experimental.pallas import tpu as pltpu
from jax.experimental.pallas import tpu_sc as plsc

D = 64
B = 16384
NC = 2
NS = 16
L = 16
NW = NC * NS
BPW = B // NW
CHUNK = 128
NCHUNK = BPW // CHUNK
N_ENT = 1_000_000
N_REL = 1000
GRP = 16000
NGRP = -(-N_ENT // GRP)
P_ROWS = ((NGRP + 1) // 2) * GRP


def _eye(dtype):
    return (lax.broadcasted_iota(jnp.int32, (D, D), 0)
            == lax.broadcasted_iota(jnp.int32, (D, D), 1)).astype(dtype)


def _transpose_pack_entities(ent_t):
    def body(x0_ref, x1_ref, o_ref):
        eye = _eye(jnp.bfloat16)
        y0 = lax.dot_general(x0_ref[...].astype(jnp.bfloat16), eye,
                             (((0,), (0,)), ((), ())),
                             preferred_element_type=jnp.float32)
        y1 = lax.dot_general(x1_ref[...].astype(jnp.bfloat16), eye,
                             (((0,), (0,)), ((), ())),
                             preferred_element_type=jnp.float32)
        o_ref[:, 0:D] = y0
        o_ref[:, D:2 * D] = y1

    return pl.pallas_call(
        body,
        grid=((NGRP + 1) // 2,),
        in_specs=[
            pl.BlockSpec((D, GRP), lambda c: (0, 2 * c)),
            pl.BlockSpec((D, GRP),
                         lambda c: (0, jnp.minimum(2 * c + 1, NGRP - 1))),
        ],
        out_specs=pl.BlockSpec((GRP, 2 * D), lambda c: (c, 0)),
        out_shape=jax.ShapeDtypeStruct((P_ROWS, 2 * D), jnp.float32),
    )(ent_t, ent_t)


def _transpose_pad_relations(rel_t):
    def body(x_ref, o_ref):
        y = lax.dot_general(x_ref[...], _eye(jnp.float32),
                            (((0,), (0,)), ((), ())),
                            preferred_element_type=jnp.float32)
        o_ref[:, 0:D] = y
        o_ref[:, D:2 * D] = jnp.zeros_like(y)

    return pl.pallas_call(
        body,
        out_shape=jax.ShapeDtypeStruct((N_REL, 2 * D), jnp.float32),
    )(rel_t)


def _sc_gather_score(subj, rel, obj, p_ent, p_rel):
    mesh = plsc.VectorSubcoreMesh(core_axis_name="c", subcore_axis_name="s")

    @functools.partial(
        pl.kernel,
        mesh=mesh,
        out_type=jax.ShapeDtypeStruct((B,), jnp.float32),
        compiler_params=pltpu.CompilerParams(needs_layout_passes=False),
        scratch_types=[
            pltpu.VMEM((BPW,), jnp.int32),
            pltpu.VMEM((BPW,), jnp.int32),
            pltpu.VMEM((BPW,), jnp.int32),
            pltpu.VMEM((BPW,), jnp.int32),
            pltpu.VMEM((BPW,), jnp.int32),
            pltpu.VMEM((2, CHUNK, 2 * D), jnp.float32),
            pltpu.VMEM((2, CHUNK, 2 * D), jnp.float32),
            pltpu.VMEM((2, CHUNK, 2 * D), jnp.float32),
            pltpu.VMEM((BPW,), jnp.float32),
            pltpu.SemaphoreType.DMA,
            pltpu.SemaphoreType.DMA,
        ],
    )
    def k(subj_h, rel_h, obj_h, ent_h, relf_h, out_h,
          sidx, ridx, oidx, soff, ooff, srows, rrows, orows, outv,
          sem0, sem1):
        wid = lax.axis_index("s") * NC + lax.axis_index("c")
        base = wid * BPW
        pltpu.sync_copy(subj_h.at[pl.ds(base, BPW)], sidx)
        pltpu.sync_copy(rel_h.at[pl.ds(base, BPW)], ridx)
        pltpu.sync_copy(obj_h.at[pl.ds(base, BPW)], oidx)

        def xform(v, idx_ref, off_ref, i):
            g = v // GRP
            p = (g >> 1) * GRP + (v - g * GRP)
            idx_ref[pl.ds(i * L, L)] = p
            off_ref[pl.ds(i * L, L)] = (g & 1) * D

        def xf_body(i, carry):
            sl = pl.ds(i * L, L)
            xform(sidx[sl], sidx, soff, i)
            xform(oidx[sl], oidx, ooff, i)
            return carry

        lax.fori_loop(0, BPW // L, xf_body, 0)

        iota = lax.iota(jnp.int32, L)
        sems = (sem0, sem1)

        def issue(c):
            isl = pl.ds(c * CHUNK, CHUNK)
            pb = c % 2
            sem = sems[pb]
            return [
                pltpu.async_copy(ent_h.at[sidx.at[isl]], srows.at[pb], sem),
                pltpu.async_copy(relf_h.at[ridx.at[isl]], rrows.at[pb], sem),
                pltpu.async_copy(ent_h.at[oidx.at[isl]], orows.at[pb], sem),
            ]

        pending = issue(0)
        for c in range(NCHUNK):
            nxt = issue(c + 1) if c + 1 < NCHUNK else None
            for cp in pending:
                cp.wait()
            pending = nxt
            pb = c % 2
            sbuf, rbuf, obuf = srows.at[pb], rrows.at[pb], orows.at[pb]

            def g_body(g, carry):
                base_p = g * L
                sofv = soff[pl.ds(c * CHUNK + base_p, L)]
                oofv = ooff[pl.ds(c * CHUNK + base_p, L)]
                acc_vec = jnp.zeros((L,), jnp.float32)
                for j in range(L):
                    p = base_p + j
                    s_hi = sofv[j] != 0
                    o_hi = oofv[j] != 0
                    acc = None
                    for kk in range(D // L):
                        sv = jnp.where(s_hi,
                                       sbuf[p, pl.ds(D + kk * L, L)],
                                       sbuf[p, pl.ds(kk * L, L)])
                        rv = rbuf[p, pl.ds(kk * L, L)]
                        ov = jnp.where(o_hi,
                                       obuf[p, pl.ds(D + kk * L, L)],
                                       obuf[p, pl.ds(kk * L, L)])
                        t = sv * rv * ov
                        acc = t if acc is None else acc + t
                    acc_vec = jnp.where(iota == j, jnp.sum(acc), acc_vec)
                outv[pl.ds(c * CHUNK + base_p, L)] = acc_vec
                return carry

            lax.fori_loop(0, CHUNK // L, g_body, 0)

        pltpu.sync_copy(outv, out_h.at[pl.ds(base, BPW)])

    return k(subj, rel, obj, p_ent, p_rel)


def kernel(subjects, relations, objects, entity_factors, relations_factors):
    p_ent = _transpose_pack_entities(entity_factors.T)
    p_rel = _transpose_pad_relations(relations_factors.T)
    return _sc_gather_score(subjects, relations, objects, p_ent, p_rel)


# speedup vs baseline: 1.3608x; 1.0006x over previous
import functools

import jax
import jax.numpy as jnp
from jax import lax
from jax.experimental import pallas as pl
from jax.experimental.pallas import tpu as pltpu
from jax.experimental.pallas import tpu_sc as plsc

D = 64
B = 16384
NC = 2
NS = 16
L = 16
NW = NC * NS
BPW = B // NW
CHUNK = 128
NCHUNK = BPW // CHUNK
N_ENT = 1_000_000
N_REL = 1000
GRP = 12800
NGRP = -(-N_ENT // GRP)
P_ROWS = ((NGRP + 3) // 4) * GRP


def _eye(dtype):
    return (lax.broadcasted_iota(jnp.int32, (D, D), 0)
            == lax.broadcasted_iota(jnp.int32, (D, D), 1)).astype(dtype)


def _sel(parity):
    return (lax.broadcasted_iota(jnp.int32, (D, D // 2), 0)
            == (2 * lax.broadcasted_iota(jnp.int32, (D, D // 2), 1) + parity)
            ).astype(jnp.bfloat16)


def _pack_quarter(x):
    xb = x.astype(jnp.bfloat16)
    dn = (((0,), (0,)), ((), ()))
    ye = lax.dot_general(xb, _sel(0), dn, preferred_element_type=jnp.float32)
    yo = lax.dot_general(xb, _sel(1), dn, preferred_element_type=jnp.float32)
    be = lax.bitcast_convert_type(ye.astype(jnp.bfloat16), jnp.uint16)
    bo = lax.bitcast_convert_type(yo.astype(jnp.bfloat16), jnp.uint16)
    word = (bo.astype(jnp.uint32) << 16) | be.astype(jnp.uint32)
    return lax.bitcast_convert_type(word, jnp.float32)


def _transpose_pack_entities(ent_t):
    def body(x0_ref, x1_ref, x2_ref, x3_ref, o_ref):
        for q, xr in enumerate((x0_ref, x1_ref, x2_ref, x3_ref)):
            o_ref[:, pl.ds(q * (D // 2), D // 2)] = _pack_quarter(xr[...])

    return pl.pallas_call(
        body,
        grid=((NGRP + 3) // 4,),
        in_specs=[
            pl.BlockSpec((D, GRP),
                         lambda c, q=q: (0, jnp.minimum(4 * c + q, NGRP - 1)))
            for q in range(4)
        ],
        out_specs=pl.BlockSpec((GRP, 2 * D), lambda c: (c, 0)),
        out_shape=jax.ShapeDtypeStruct((P_ROWS, 2 * D), jnp.float32),
    )(ent_t, ent_t, ent_t, ent_t)


def _transpose_pad_relations(rel_t):
    def body(x_ref, o_ref):
        z = _pack_quarter(x_ref[...])
        o_ref[...] = jnp.concatenate(
            [z, jnp.zeros((N_REL, 3 * (D // 2)), jnp.float32)], axis=1)

    return pl.pallas_call(
        body,
        out_shape=jax.ShapeDtypeStruct((N_REL, 2 * D), jnp.float32),
    )(rel_t)


def _sc_gather_score(subj, rel, obj, p_ent, p_rel):
    mesh = plsc.VectorSubcoreMesh(core_axis_name="c", subcore_axis_name="s")

    @functools.partial(
        pl.kernel,
        mesh=mesh,
        out_type=jax.ShapeDtypeStruct((B,), jnp.float32),
        compiler_params=pltpu.CompilerParams(needs_layout_passes=False),
        scratch_types=[
            pltpu.VMEM((BPW,), jnp.int32),
            pltpu.VMEM((BPW,), jnp.int32),
            pltpu.VMEM((BPW,), jnp.int32),
            pltpu.VMEM((BPW,), jnp.int32),
            pltpu.VMEM((BPW,), jnp.int32),
            pltpu.VMEM((2, CHUNK, 2 * D), jnp.float32),
            pltpu.VMEM((2, CHUNK, 2 * D), jnp.float32),
            pltpu.VMEM((2, CHUNK, 2 * D), jnp.float32),
            pltpu.VMEM((BPW,), jnp.float32),
            pltpu.SemaphoreType.DMA,
            pltpu.SemaphoreType.DMA,
        ],
    )
    def k(subj_h, rel_h, obj_h, ent_h, relf_h, out_h,
          sidx, ridx, oidx, soff, ooff, srows, rrows, orows, outv,
          sem0, sem1):
        wid = lax.axis_index("s") * NC + lax.axis_index("c")
        base = wid * BPW
        pltpu.sync_copy(subj_h.at[pl.ds(base, BPW)], sidx)
        pltpu.sync_copy(rel_h.at[pl.ds(base, BPW)], ridx)
        pltpu.sync_copy(obj_h.at[pl.ds(base, BPW)], oidx)

        def xform(v, idx_ref, off_ref, i):
            g = v // GRP
            p = (g >> 2) * GRP + (v - g * GRP)
            idx_ref[pl.ds(i * L, L)] = p
            off_ref[pl.ds(i * L, L)] = (g & 3) * (D // 2)

        def xf_body(i, carry):
            sl = pl.ds(i * L, L)
            xform(sidx[sl], sidx, soff, i)
            xform(oidx[sl], oidx, ooff, i)
            return carry

        lax.fori_loop(0, BPW // L, xf_body, 0)

        iota = lax.iota(jnp.int32, L)
        sems = (sem0, sem1)

        def issue(c):
            isl = pl.ds(c * CHUNK, CHUNK)
            pb = c % 2
            sem = sems[pb]
            return [
                pltpu.async_copy(ent_h.at[sidx.at[isl]], srows.at[pb], sem),
                pltpu.async_copy(relf_h.at[ridx.at[isl]], rrows.at[pb], sem),
                pltpu.async_copy(ent_h.at[oidx.at[isl]], orows.at[pb], sem),
            ]

        pending = issue(0)
        for c in range(NCHUNK):
            nxt = issue(c + 1) if c + 1 < NCHUNK else None
            for cp in pending:
                cp.wait()
            pending = nxt
            pb = c % 2
            sbuf, rbuf, obuf = srows.at[pb], rrows.at[pb], orows.at[pb]

            def g_body(g, carry):
                base_p = g * L
                sofv = soff[pl.ds(c * CHUNK + base_p, L)]
                oofv = ooff[pl.ds(c * CHUNK + base_p, L)]
                acc_vec = jnp.zeros((L,), jnp.float32)
                for j in range(L):
                    p = base_p + j
                    s_hi = sofv[j] >= D
                    s_odd = (sofv[j] & (D // 2)) != 0
                    o_hi = oofv[j] >= D
                    o_odd = (oofv[j] & (D // 2)) != 0

                    def pick(buf, hi, odd, base_w, p=p):
                        lo = jnp.where(odd,
                                       buf[p, pl.ds(32 + base_w, L)],
                                       buf[p, pl.ds(base_w, L)])
                        hh = jnp.where(odd,
                                       buf[p, pl.ds(96 + base_w, L)],
                                       buf[p, pl.ds(64 + base_w, L)])
                        return jnp.where(hi, hh, lo)

                    def to_bf(w):
                        return plsc.unpack(
                            plsc.bitcast(w, jnp.bfloat16),
                            format=plsc.PackFormat.INTERLEAVED,
                            preferred_element_type=jnp.float32)

                    acc = None
                    for kk in range(2):
                        bw = kk * L
                        sa, sb2 = to_bf(pick(sbuf, s_hi, s_odd, bw))
                        ra, rb2 = to_bf(rbuf[p, pl.ds(bw, L)])
                        oa, ob2 = to_bf(pick(obuf, o_hi, o_odd, bw))
                        t = sa * ra * oa + sb2 * rb2 * ob2
                        acc = t if acc is None else acc + t
                    acc_vec = jnp.where(iota == j, jnp.sum(acc), acc_vec)
                outv[pl.ds(c * CHUNK + base_p, L)] = acc_vec
                return carry

            lax.fori_loop(0, CHUNK // L, g_body, 0)

        pltpu.sync_copy(outv, out_h.at[pl.ds(base, BPW)])

    return k(subj, rel, obj, p_ent, p_rel)


def kernel(subjects, relations, objects, entity_factors, relations_factors):
    p_ent = _transpose_pack_entities(entity_factors.T)
    p_rel = _transpose_pad_relations(relations_factors.T)
    return _sc_gather_score(subjects, relations, objects, p_ent, p_rel)
